# baseline (speedup 1.0000x reference)
import jax
import jax.numpy as jnp
from jax.experimental import pallas as pl

_BLOCK = 512


def _body(x_ref, We1_ref, be1_ref, We2_ref, be2_ref, We3_ref, be3_ref,
          cb_ref, Wd1_ref, bd1_ref, Wd2_ref, bd2_ref, Wd3_ref, bd3_ref,
          xr_ref, zq_ref, sq_ref, rq_ref):
    xb = x_ref[...]
    z1 = jnp.maximum(jnp.dot(xb, We1_ref[...]) + be1_ref[...][None, :], 0.0)
    z2 = jnp.maximum(jnp.dot(z1, We2_ref[...]) + be2_ref[...][None, :], 0.0)
    z = jnp.dot(z2, We3_ref[...]) + be3_ref[...][None, :]
    cb = cb_ref[...]
    d = (jnp.sum(z * z, axis=1, keepdims=True)
         + jnp.sum(cb * cb, axis=1)[None, :]
         - 2.0 * jnp.dot(z, cb.T))
    idx = jnp.argmin(d, axis=1)
    onehot = (jax.lax.broadcasted_iota(jnp.int32, d.shape, 1)
              == idx[:, None]).astype(jnp.float32)
    zq = jnp.dot(onehot, cb, precision=jax.lax.Precision.HIGHEST)
    h = jnp.maximum(jnp.dot(zq, Wd1_ref[...]) + bd1_ref[...][None, :], 0.0)
    h = jnp.maximum(jnp.dot(h, Wd2_ref[...]) + bd2_ref[...][None, :], 0.0)
    xr = jnp.dot(h, Wd3_ref[...]) + bd3_ref[...][None, :]

    xr_ref[...] = xr
    zq_ref[...] = zq

    sq = jnp.sum((zq - z) ** 2)
    rq = jnp.sum((xr - xb) ** 2)
    first = pl.program_id(0) == 0
    sq_ref[...] = jnp.where(first, 0.0, sq_ref[...]) + sq
    rq_ref[...] = jnp.where(first, 0.0, rq_ref[...]) + rq


def kernel(x, We1, be1, We2, be2, We3, be3, codebook,
           Wd1, bd1, Wd2, bd2, Wd3, bd3):
    n, d_in = x.shape
    h_dim = We1.shape[1]
    l_dim = We3.shape[1]
    k_dim = codebook.shape[0]
    blk = _BLOCK
    grid = n // blk

    full = lambda a: pl.BlockSpec(a.shape, lambda i: (0,) * a.ndim)
    out_shapes = (
        jax.ShapeDtypeStruct((n, d_in), jnp.float32),
        jax.ShapeDtypeStruct((n, l_dim), jnp.float32),
        jax.ShapeDtypeStruct((1, 1), jnp.float32),
        jax.ShapeDtypeStruct((1, 1), jnp.float32),
    )
    xr, zq, sqs, rqs = pl.pallas_call(
        _body,
        grid=(grid,),
        in_specs=[
            pl.BlockSpec((blk, d_in), lambda i: (i, 0)),
            full(We1), full(be1), full(We2), full(be2), full(We3), full(be3),
            full(codebook), full(Wd1), full(bd1), full(Wd2), full(bd2),
            full(Wd3), full(bd3),
        ],
        out_specs=(
            pl.BlockSpec((blk, d_in), lambda i: (i, 0)),
            pl.BlockSpec((blk, l_dim), lambda i: (i, 0)),
            pl.BlockSpec((1, 1), lambda i: (0, 0)),
            pl.BlockSpec((1, 1), lambda i: (0, 0)),
        ),
        out_shape=out_shapes,
    )(x, We1, be1, We2, be2, We3, be3, codebook, Wd1, bd1, Wd2, bd2, Wd3, bd3)

    vq_loss = 1.25 * sqs[0, 0] / (n * l_dim)
    recon_loss = rqs[0, 0] / (n * d_in)
    total_loss = recon_loss + vq_loss
    return (xr, total_loss, vq_loss, zq)


# speedup vs baseline: 1.5251x; 1.5251x over previous
import jax
import jax.numpy as jnp
from jax.experimental import pallas as pl

_BLOCK = 512


def _body(x_ref, We1_ref, be1_ref, We2_ref, be2_ref, We3_ref, be3_ref,
          cb_ref, Wd1_ref, bd1_ref, Wd2_ref, bd2_ref, Wd3_ref, bd3_ref,
          xr_ref, zq_ref, sq_ref, rq_ref):
    xb = x_ref[...]
    z1 = jnp.maximum(jnp.dot(xb, We1_ref[...]) + be1_ref[...][None, :], 0.0)
    z2 = jnp.maximum(jnp.dot(z1, We2_ref[...]) + be2_ref[...][None, :], 0.0)
    z = jnp.dot(z2, We3_ref[...]) + be3_ref[...][None, :]
    cb = cb_ref[...]
    d = (jnp.sum(z * z, axis=1, keepdims=True)
         + jnp.sum(cb * cb, axis=1)[None, :]
         - 2.0 * jnp.dot(z, cb.T))
    idx = jnp.argmin(d, axis=1)
    onehot = (jax.lax.broadcasted_iota(jnp.int32, d.shape, 1)
              == idx[:, None]).astype(jnp.float32)
    zq = jnp.dot(onehot, cb)
    h = jnp.maximum(jnp.dot(zq, Wd1_ref[...]) + bd1_ref[...][None, :], 0.0)
    h = jnp.maximum(jnp.dot(h, Wd2_ref[...]) + bd2_ref[...][None, :], 0.0)
    xr = jnp.dot(h, Wd3_ref[...]) + bd3_ref[...][None, :]

    xr_ref[...] = xr
    zq_ref[...] = zq

    sq = jnp.sum((zq - z) ** 2)
    rq = jnp.sum((xr - xb) ** 2)
    first = pl.program_id(0) == 0
    sq_ref[...] = jnp.where(first, 0.0, sq_ref[...]) + sq
    rq_ref[...] = jnp.where(first, 0.0, rq_ref[...]) + rq


def kernel(x, We1, be1, We2, be2, We3, be3, codebook,
           Wd1, bd1, Wd2, bd2, Wd3, bd3):
    n, d_in = x.shape
    h_dim = We1.shape[1]
    l_dim = We3.shape[1]
    k_dim = codebook.shape[0]
    blk = _BLOCK
    grid = n // blk

    full = lambda a: pl.BlockSpec(a.shape, lambda i: (0,) * a.ndim)
    out_shapes = (
        jax.ShapeDtypeStruct((n, d_in), jnp.float32),
        jax.ShapeDtypeStruct((n, l_dim), jnp.float32),
        jax.ShapeDtypeStruct((1, 1), jnp.float32),
        jax.ShapeDtypeStruct((1, 1), jnp.float32),
    )
    xr, zq, sqs, rqs = pl.pallas_call(
        _body,
        grid=(grid,),
        in_specs=[
            pl.BlockSpec((blk, d_in), lambda i: (i, 0)),
            full(We1), full(be1), full(We2), full(be2), full(We3), full(be3),
            full(codebook), full(Wd1), full(bd1), full(Wd2), full(bd2),
            full(Wd3), full(bd3),
        ],
        out_specs=(
            pl.BlockSpec((blk, d_in), lambda i: (i, 0)),
            pl.BlockSpec((blk, l_dim), lambda i: (i, 0)),
            pl.BlockSpec((1, 1), lambda i: (0, 0)),
            pl.BlockSpec((1, 1), lambda i: (0, 0)),
        ),
        out_shape=out_shapes,
    )(x, We1, be1, We2, be2, We3, be3, codebook, Wd1, bd1, Wd2, bd2, Wd3, bd3)

    vq_loss = 1.25 * sqs[0, 0] / (n * l_dim)
    recon_loss = rqs[0, 0] / (n * d_in)
    total_loss = recon_loss + vq_loss
    return (xr, total_loss, vq_loss, zq)
